# baseline (speedup 1.0000x reference)
import functools

import jax
import jax.numpy as jnp
from jax import lax
from jax.experimental import pallas as pl
from jax.experimental.pallas import tpu as pltpu
from jax.experimental.pallas import tpu_sc as plsc

N = 10000
E = 320000
D = 128
COLS = D // 2
NS = 16
ROWS_PER_TILE = N // NS
CHUNK = 128
NCHUNK = -(-E // (NS * CHUNK) // 2) * 2
E_PAD = NS * NCHUNK * CHUNK
NROWS = N + 8


def _sc_aggregate(x2, src3, dst3):
  mesh = plsc.VectorSubcoreMesh(core_axis_name="c", subcore_axis_name="s")

  @functools.partial(
      pl.kernel,
      mesh=mesh,
      compiler_params=pltpu.CompilerParams(use_tc_tiling_on_sc=False),
      out_type=jax.ShapeDtypeStruct((N, D), jnp.float32),
      scratch_types=[
          pltpu.VMEM_SHARED((NROWS, COLS), jnp.float32),
          pltpu.VMEM((NCHUNK, CHUNK), jnp.int32),
          pltpu.VMEM((NCHUNK, CHUNK), jnp.int32),
          pltpu.VMEM((CHUNK, COLS), jnp.float32),
          pltpu.VMEM((CHUNK, COLS), jnp.float32),
          pltpu.SemaphoreType.DMA,
          pltpu.SemaphoreType.DMA,
          pltpu.SemaphoreType.DMA,
          pltpu.SemaphoreType.DMA,
      ],
  )
  def k(x2_hbm, src_hbm, dst_hbm, h_hbm, agg_s, src_v, dst_v,
        rows0, rows1, gsem0, gsem1, ssem0, ssem1):
    c = lax.axis_index("c")
    s = lax.axis_index("s")
    r0 = s * ROWS_PER_TILE
    c0 = c * COLS
    x_half = x2_hbm.at[c]
    pltpu.sync_copy(x_half.at[pl.ds(r0, ROWS_PER_TILE)],
                    agg_s.at[pl.ds(r0, ROWS_PER_TILE)])
    pltpu.sync_copy(src_hbm.at[s], src_v)
    pltpu.sync_copy(dst_hbm.at[s], dst_v)
    plsc.subcore_barrier()

    bufs = (rows0, rows1)
    gsems = (gsem0, gsem1)
    ssems = (ssem0, ssem1)

    def gather(j, b):
      pltpu.async_copy(x_half.at[src_v.at[j]], bufs[b], gsems[b])

    def wait_gather(b):
      pltpu.make_async_copy(x_half.at[src_v.at[0]], bufs[b], gsems[b]).wait()

    def scatter(j, b):
      pltpu.async_copy(bufs[b], agg_s.at[dst_v.at[j]], ssems[b], add=True)

    def wait_scatter(b):
      pltpu.make_async_copy(bufs[b], agg_s.at[dst_v.at[0]], ssems[b]).wait()

    gather(0, 0)
    gather(1, 1)

    def step(jj, carry):
      for b in range(2):
        j = 2 * jj + b
        wait_gather(b)
        scatter(j, b)
        wait_scatter(b)
        gather(j + 2, b)
      return carry

    lax.fori_loop(0, NCHUNK // 2 - 1, step, 0)
    for j in (NCHUNK - 2, NCHUNK - 1):
      b = j % 2
      wait_gather(b)
      scatter(j, b)
      wait_scatter(b)

    plsc.subcore_barrier()
    pltpu.sync_copy(agg_s.at[pl.ds(r0, ROWS_PER_TILE)],
                    h_hbm.at[pl.ds(r0, ROWS_PER_TILE), pl.ds(c0, COLS)])

  return k(x2, src3, dst3)


def _mlp_body(h_ref, w1_ref, b1_ref, w2_ref, b2_ref, o_ref):
  h = h_ref[...]
  a = jnp.dot(h, w1_ref[...], preferred_element_type=jnp.float32) + b1_ref[...]
  a = jnp.maximum(a, 0.0)
  o_ref[...] = jnp.dot(a, w2_ref[...], preferred_element_type=jnp.float32) + b2_ref[...]


def _mlp(h, W1, b1, W2, b2):
  blk = 1000
  return pl.pallas_call(
      _mlp_body,
      grid=(N // blk,),
      in_specs=[
          pl.BlockSpec((blk, D), lambda i: (i, 0)),
          pl.BlockSpec((D, D), lambda i: (0, 0)),
          pl.BlockSpec((1, D), lambda i: (0, 0)),
          pl.BlockSpec((D, D), lambda i: (0, 0)),
          pl.BlockSpec((1, D), lambda i: (0, 0)),
      ],
      out_specs=pl.BlockSpec((blk, D), lambda i: (i, 0)),
      out_shape=jax.ShapeDtypeStruct((N, D), jnp.float32),
  )(h, W1, b1, W2, b2)


def kernel(x, edge_index, W1, b1, W2, b2):
  npad = E_PAD - E
  src = jnp.concatenate([edge_index[0], jnp.zeros((npad,), jnp.int32)])
  dst = jnp.concatenate([edge_index[1], jnp.full((npad,), N, jnp.int32)])
  src3 = src.reshape(NS, NCHUNK, CHUNK)
  dst3 = dst.reshape(NS, NCHUNK, CHUNK)
  x2 = jnp.stack([x[:, :COLS], x[:, COLS:]])
  h = _sc_aggregate(x2, src3, dst3)
  return _mlp(h, W1, b1.reshape(1, D), W2, b2.reshape(1, D))
